# baseline (speedup 1.0000x reference)
import functools

import jax
import jax.numpy as jnp
from jax import lax
from jax.experimental import pallas as pl
from jax.experimental.pallas import tpu as pltpu
from jax.experimental.pallas import tpu_sc as plsc

E = 320000
D = 128
V = 20

NW = 32
PER_W = E // NW
GROUP = 128
NBUF = 5
BATCH = GROUP * NBUF
NOUTER = 15
TAIL_G = 3
TAIL_R = PER_W - (NOUTER * BATCH + TAIL_G * GROUP)
FUSE_W = PER_W // 16


def _combos_body(e0_ref, e1_ref, e2_ref, out_ref):
    t01 = e0_ref[...][:, None, None, :] + e1_ref[...][None, :, None, :]
    blk = t01 + e2_ref[...][None, None, :, :]
    out_ref[...] = blk.reshape(V * V * V, D)


def _combos(emb0, emb1, emb2):
    return pl.pallas_call(
        _combos_body,
        out_shape=jax.ShapeDtypeStruct((V * V * V, D), jnp.float32),
    )(emb0, emb1, emb2)


@functools.partial(
    pl.kernel,
    mesh=plsc.VectorSubcoreMesh(core_axis_name="c", subcore_axis_name="s"),
    out_type=jax.ShapeDtypeStruct((E, D), jnp.float32),
    scratch_types=(
        [pltpu.VMEM((PER_W,), jnp.int32)] * 4
        + [pltpu.VMEM((GROUP, D), jnp.float32)] * NBUF
        + [pltpu.SemaphoreType.DMA] * (1 + 2 * NBUF)
    ),
)
def _sc_gather(attr_hbm, combos_hbm, out_hbm,
               a0_v, a1_v, a2_v, cidx_v, *bufs_and_sems):
    rows = bufs_and_sems[:NBUF]
    isem = bufs_and_sems[NBUF]
    gsem = bufs_and_sems[NBUF + 1:2 * NBUF + 1]
    ssem = bufs_and_sems[2 * NBUF + 1:]
    wid = lax.axis_index("s") * 2 + lax.axis_index("c")
    base = wid * PER_W

    cps = [pltpu.async_copy(attr_hbm.at[pl.ds(k * E + base, PER_W)],
                            v, isem)
           for k, v in ((0, a0_v), (1, a1_v), (2, a2_v))]
    for cp in cps:
        cp.wait()

    def fuse_range(lo, hi):
        def fuse_body(j, carry):
            sl = pl.ds(pl.multiple_of(j * 16, 16), 16)
            cidx_v[sl] = a0_v[sl] * 400 + a1_v[sl] * 20 + a2_v[sl]
            return carry
        lax.fori_loop(lo, hi, fuse_body, 0)

    fuse_range(0, BATCH // 16)

    def outer_body(o, carry):
        goff = pl.multiple_of(o * BATCH, BATCH)
        gcps = []
        for b in range(NBUF):
            @pl.when(o > 0)
            def _(b=b):
                pltpu.make_async_copy(
                    rows[b], out_hbm.at[pl.ds(0, GROUP)], ssem[b]).wait()
            cidx_sl = cidx_v.at[pl.ds(goff + b * GROUP, GROUP)]
            gcps.append(pltpu.async_copy(
                combos_hbm.at[cidx_sl], rows[b], gsem[b]))
        lo = (o + 1) * (BATCH // 16)
        fuse_range(jnp.minimum(lo, FUSE_W),
                   jnp.minimum(lo + BATCH // 16, FUSE_W))
        for b in range(NBUF):
            gcps[b].wait()
            pltpu.async_copy(
                rows[b], out_hbm.at[pl.ds(base + goff + b * GROUP, GROUP)],
                ssem[b])
        return carry

    lax.fori_loop(0, NOUTER, outer_body, 0)

    eoff = NOUTER * BATCH
    egcps = []
    for b in range(TAIL_G):
        pltpu.make_async_copy(
            rows[b], out_hbm.at[pl.ds(0, GROUP)], ssem[b]).wait()
        egcps.append(pltpu.async_copy(
            combos_hbm.at[cidx_v.at[pl.ds(eoff + b * GROUP, GROUP)]],
            rows[b], gsem[b]))
    pltpu.make_async_copy(
        rows[TAIL_G], out_hbm.at[pl.ds(0, GROUP)], ssem[TAIL_G]).wait()
    toff = eoff + TAIL_G * GROUP
    tcp = pltpu.async_copy(
        combos_hbm.at[cidx_v.at[pl.ds(toff, TAIL_R)]],
        rows[TAIL_G].at[pl.ds(0, TAIL_R)], gsem[TAIL_G])
    for b in range(TAIL_G):
        egcps[b].wait()
        pltpu.async_copy(
            rows[b], out_hbm.at[pl.ds(base + eoff + b * GROUP, GROUP)],
            ssem[b])
    tcp.wait()
    pltpu.async_copy(rows[TAIL_G].at[pl.ds(0, TAIL_R)],
                     out_hbm.at[pl.ds(base + toff, TAIL_R)], ssem[TAIL_G])
    pltpu.make_async_copy(
        rows[NBUF - 1], out_hbm.at[pl.ds(0, GROUP)], ssem[NBUF - 1]).wait()
    for b in range(TAIL_G):
        pltpu.make_async_copy(
            rows[b], out_hbm.at[pl.ds(0, GROUP)], ssem[b]).wait()
    pltpu.make_async_copy(
        rows[TAIL_G].at[pl.ds(0, TAIL_R)],
        out_hbm.at[pl.ds(0, TAIL_R)], ssem[TAIL_G]).wait()


def kernel(edge_attr, emb0, emb1, emb2):
    combos = _combos(emb0, emb1, emb2)
    return _sc_gather(edge_attr.T.reshape(3 * E), combos)


# speedup vs baseline: 1.0072x; 1.0072x over previous
import functools

import jax
import jax.numpy as jnp
from jax import lax
from jax.experimental import pallas as pl
from jax.experimental.pallas import tpu as pltpu
from jax.experimental.pallas import tpu_sc as plsc

E = 320000
D = 128
V = 20

NW = 32
PER_W = E // NW
GROUP = 80
NBUF = 5
BATCH = GROUP * NBUF
NOUTER = PER_W // BATCH


def _combos_body(e0_ref, e1_ref, e2_ref, out_ref):
    t01 = e0_ref[...][:, None, None, :] + e1_ref[...][None, :, None, :]
    blk = t01 + e2_ref[...][None, None, :, :]
    out_ref[...] = blk.reshape(V * V * V, D)


def _combos(emb0, emb1, emb2):
    return pl.pallas_call(
        _combos_body,
        out_shape=jax.ShapeDtypeStruct((V * V * V, D), jnp.float32),
    )(emb0, emb1, emb2)


@functools.partial(
    pl.kernel,
    mesh=plsc.VectorSubcoreMesh(core_axis_name="c", subcore_axis_name="s"),
    out_type=jax.ShapeDtypeStruct((E, D), jnp.float32),
    scratch_types=(
        [pltpu.VMEM((PER_W,), jnp.int32)] * 4
        + [pltpu.VMEM((GROUP, D), jnp.float32)] * NBUF
        + [pltpu.SemaphoreType.DMA] * (1 + 2 * NBUF)
    ),
)
def _sc_gather(attr_hbm, combos_hbm, out_hbm,
               a0_v, a1_v, a2_v, cidx_v, *bufs_and_sems):
    rows = bufs_and_sems[:NBUF]
    isem = bufs_and_sems[NBUF]
    gsem = bufs_and_sems[NBUF + 1:2 * NBUF + 1]
    ssem = bufs_and_sems[2 * NBUF + 1:]
    wid = lax.axis_index("s") * 2 + lax.axis_index("c")
    base = wid * PER_W

    cps = [pltpu.async_copy(attr_hbm.at[pl.ds(k * E + base, PER_W)],
                            v, isem)
           for k, v in ((0, a0_v), (1, a1_v), (2, a2_v))]
    for cp in cps:
        cp.wait()

    def fuse_batch(o):
        def fuse_body(j, carry):
            sl = pl.ds(pl.multiple_of(j * 16, 16), 16)
            cidx_v[sl] = a0_v[sl] * 400 + a1_v[sl] * 20 + a2_v[sl]
            return carry
        lax.fori_loop(o * (BATCH // 16), (o + 1) * (BATCH // 16),
                      fuse_body, 0)

    fuse_batch(0)

    def outer_body(o, carry):
        goff = pl.multiple_of(o * BATCH, BATCH)
        gcps = []
        for b in range(NBUF):
            @pl.when(o > 0)
            def _(b=b):
                pltpu.make_async_copy(
                    rows[b], out_hbm.at[pl.ds(0, GROUP)], ssem[b]).wait()
            cidx_sl = cidx_v.at[pl.ds(goff + b * GROUP, GROUP)]
            gcps.append(pltpu.async_copy(
                combos_hbm.at[cidx_sl], rows[b], gsem[b]))
        @pl.when(o + 1 < NOUTER)
        def _():
            fuse_batch(o + 1)
        for b in range(NBUF):
            gcps[b].wait()
            pltpu.async_copy(
                rows[b], out_hbm.at[pl.ds(base + goff + b * GROUP, GROUP)],
                ssem[b])
        return carry

    lax.fori_loop(0, NOUTER, outer_body, 0)
    for b in range(NBUF):
        pltpu.make_async_copy(
            rows[b], out_hbm.at[pl.ds(0, GROUP)], ssem[b]).wait()


def kernel(edge_attr, emb0, emb1, emb2):
    combos = _combos(emb0, emb1, emb2)
    return _sc_gather(edge_attr.T.reshape(3 * E), combos)
